# Initial kernel scaffold; baseline (speedup 1.0000x reference)
import jax
import jax.numpy as jnp
from jax.experimental import pallas as pl


def kernel(decoder_hidden, context_vector, encoder_outputs, attention_weights, vocab_distribution, source_chars, W1, b1, W2, b2):
    raise NotImplementedError("write your pallas kernel here")


# speedup vs baseline: 7.4247x; 7.4247x over previous
import functools

import jax
import jax.numpy as jnp
from jax import lax
from jax.experimental import pallas as pl
from jax.experimental.pallas import tpu as pltpu
from jax.experimental.pallas import tpu_sc as plsc

_B = 1024
_SRC = 200
_DEC = 512
_ENC = 512
_V = 1000

_L = 16
_NC = 2
_NS = 16
_NW = _NC * _NS
_GPW = _B // _L // _NW
_VP = 1024
_SP = 256
_VCH = _VP // _L
_SCH = _SP // _L


def _gate_body(dh_ref, cv_ref, w1_ref, b1_ref, w2_ref, b2_ref, p_ref):
    w1 = w1_ref[...]
    h = lax.dot_general(dh_ref[...], w1[:, :_DEC], (((1,), (1,)), ((), ())),
                        preferred_element_type=jnp.float32)
    h += lax.dot_general(cv_ref[...], w1[:, _DEC:], (((1,), (1,)), ((), ())),
                         preferred_element_type=jnp.float32)
    h = jnp.tanh(h + b1_ref[...])
    z = lax.dot_general(h, w2_ref[...], (((1,), (1,)), ((), ())),
                        preferred_element_type=jnp.float32)
    p_ref[...] = jax.nn.sigmoid(z[:, :1] + b2_ref[0, 0])


def _sc_body(attn_hbm, vocab_hbm, chars_hbm, p_hbm, out_hbm,
             vocab_v, acc_v, out_v, attn_v, chars_v, p_v):
    wid = lax.axis_index("s") * _NC + lax.axis_index("c")
    iota = lax.iota(jnp.int32, _L)
    zeros = jnp.zeros((_L,), jnp.float32)

    for r in range(_L):
        def _zero(i, c, r=r):
            for u in range(8):
                acc_v[r, pl.ds((i * 8 + u) * _L, _L)] = zeros
            return c
        lax.fori_loop(0, _VCH // 8, _zero, 0)

    for k in range(_GPW):
        base = (wid * _GPW + k) * _L
        pltpu.sync_copy(vocab_hbm.at[pl.ds(base, _L), :], vocab_v)
        pltpu.sync_copy(attn_hbm.at[pl.ds(base, _L), :], attn_v)
        pltpu.sync_copy(chars_hbm.at[pl.ds(base, _L), :], chars_v)
        pltpu.sync_copy(p_hbm.at[pl.ds(base, _L)], p_v)

        def _scat(s, c):
            col = jnp.full((_L,), s, jnp.int32)
            ch = plsc.load_gather(chars_v, [iota, col])
            aw = plsc.load_gather(attn_v, [iota, col])
            plsc.addupdate_scatter(acc_v, [iota, ch], aw)
            return c
        lax.fori_loop(0, _SRC, _scat, 0)

        def _row(r, c):
            r_idx = jnp.full((_L,), r, jnp.int32)
            pr = plsc.load_gather(p_v, [r_idx])
            one_m_p = 1.0 - pr

            def _vs(i, acc):
                return acc + vocab_v[r, pl.ds(i * _L, _L)]
            vsum_vec = lax.fori_loop(0, _VCH, _vs, zeros, unroll=8)
            vsum = jnp.broadcast_to(jnp.sum(vsum_vec), (_L,))

            def _as(i, acc):
                return acc + attn_v[r, pl.ds(i * _L, _L)]
            asum_vec = lax.fori_loop(0, _SCH, _as, zeros, unroll=8)
            asum = jnp.broadcast_to(jnp.sum(asum_vec), (_L,))

            total = one_m_p * vsum + pr * asum
            inv = 1.0 / (total + 1e-10)
            gs = one_m_p * inv
            ps = pr * inv

            def _fin(i, c2):
                v0 = i * _L
                vchunk = vocab_v[r, pl.ds(v0, _L)]
                achunk = acc_v[r, pl.ds(v0, _L)]
                out_v[r, pl.ds(v0, _L)] = vchunk * gs + achunk * ps
                acc_v[r, pl.ds(v0, _L)] = zeros
                return c2
            lax.fori_loop(0, _VCH, _fin, 0, unroll=8)
            return c
        lax.fori_loop(0, _L, _row, 0)

        pltpu.sync_copy(out_v, out_hbm.at[pl.ds(base, _L), :])


def kernel(decoder_hidden, context_vector, encoder_outputs, attention_weights,
           vocab_distribution, source_chars, W1, b1, W2, b2):
    del encoder_outputs

    copy_prob = pl.pallas_call(
        _gate_body,
        out_shape=jax.ShapeDtypeStruct((_B, 1), jnp.float32),
    )(decoder_hidden, context_vector, W1,
      b1.reshape(1, _DEC), jnp.pad(W2, ((0, 127), (0, 0))), b2.reshape(1, 1))

    p_flat = copy_prob.reshape(_B)
    attn_p = jnp.pad(attention_weights, ((0, 0), (0, _SP - _SRC)))
    vocab_p = jnp.pad(vocab_distribution, ((0, 0), (0, _VP - _V)))
    chars_p = jnp.pad(source_chars.astype(jnp.int32),
                      ((0, 0), (0, _SP - _SRC)))

    mesh = plsc.VectorSubcoreMesh(core_axis_name="c", subcore_axis_name="s")
    sc_call = functools.partial(
        pl.kernel, mesh=mesh,
        compiler_params=pltpu.CompilerParams(use_tc_tiling_on_sc=False,
                                             needs_layout_passes=False),
        out_type=jax.ShapeDtypeStruct((_B, _VP), jnp.float32),
        scratch_types=[
            pltpu.VMEM((_L, _VP), jnp.float32),
            pltpu.VMEM((_L, _VP), jnp.float32),
            pltpu.VMEM((_L, _VP), jnp.float32),
            pltpu.VMEM((_L, _SP), jnp.float32),
            pltpu.VMEM((_L, _SP), jnp.int32),
            pltpu.VMEM((_L,), jnp.float32),
        ],
    )(_sc_body)
    final_p = sc_call(attn_p, vocab_p, chars_p, p_flat)
    return final_p[:, :_V], copy_prob


# speedup vs baseline: 8.8157x; 1.1873x over previous
import functools

import jax
import jax.numpy as jnp
from jax import lax
from jax.experimental import pallas as pl
from jax.experimental.pallas import tpu as pltpu
from jax.experimental.pallas import tpu_sc as plsc

_B = 1024
_SRC = 200
_DEC = 512
_ENC = 512
_V = 1000

_L = 16
_NC = 2
_NS = 16
_NW = _NC * _NS
_GPW = _B // _L // _NW
_VP = 1024
_SP = 256
_VCH = _VP // _L


def _gate_body(dh_ref, cv_ref, w1_ref, b1_ref, w2_ref, b2_ref, p_ref):
    w1 = w1_ref[...]
    h = lax.dot_general(dh_ref[...], w1[:, :_DEC], (((1,), (1,)), ((), ())),
                        preferred_element_type=jnp.float32)
    h += lax.dot_general(cv_ref[...], w1[:, _DEC:], (((1,), (1,)), ((), ())),
                         preferred_element_type=jnp.float32)
    h = jnp.tanh(h + b1_ref[...])
    z = lax.dot_general(h, w2_ref[...], (((1,), (1,)), ((), ())),
                        preferred_element_type=jnp.float32)
    p_ref[...] = jax.nn.sigmoid(z[:, :1] + b2_ref[0, 0])


def _sc_body(attn_hbm, vocab_hbm, chars_hbm, p_hbm, out_hbm,
             vocab_v, attn_v, chars_v, p_v):
    wid = lax.axis_index("s") * _NC + lax.axis_index("c")
    iota = lax.iota(jnp.int32, _L)
    zeros = jnp.zeros((_L,), jnp.float32)

    for k in range(_GPW):
        base = (wid * _GPW + k) * _L
        pltpu.sync_copy(vocab_hbm.at[pl.ds(base, _L), :], vocab_v)
        pltpu.sync_copy(attn_hbm.at[pl.ds(base, _L), :], attn_v)
        pltpu.sync_copy(chars_hbm.at[pl.ds(base, _L), :], chars_v)
        pltpu.sync_copy(p_hbm.at[pl.ds(base, _L)], p_v)

        pv = p_v[...]
        ratio = pv / (1.0 - pv)

        def _scat(s, c):
            col = jnp.full((_L,), s, jnp.int32)
            ch = plsc.load_gather(chars_v, [iota, col])
            aw = plsc.load_gather(attn_v, [iota, col])
            plsc.addupdate_scatter(vocab_v, [iota, ch], aw * ratio)
            return c
        lax.fori_loop(0, _SRC, _scat, 0, unroll=4)

        def _row(r, c):
            r_idx = jnp.full((_L,), r, jnp.int32)
            pr = plsc.load_gather(p_v, [r_idx])
            one_m_p = 1.0 - pr

            def _ms(i, acc):
                return acc + vocab_v[r, pl.ds(i * _L, _L)]
            msum_vec = lax.fori_loop(0, _VCH, _ms, zeros, unroll=8)
            msum = jnp.broadcast_to(jnp.sum(msum_vec), (_L,))
            gs = one_m_p / (one_m_p * msum + 1e-10)

            def _fin(i, c2):
                sl = pl.ds(i * _L, _L)
                vocab_v[r, sl] = vocab_v[r, sl] * gs
                return c2
            lax.fori_loop(0, _VCH, _fin, 0, unroll=8)
            return c
        lax.fori_loop(0, _L, _row, 0)

        pltpu.sync_copy(vocab_v, out_hbm.at[pl.ds(base, _L), :])


def kernel(decoder_hidden, context_vector, encoder_outputs, attention_weights,
           vocab_distribution, source_chars, W1, b1, W2, b2):
    del encoder_outputs

    copy_prob = pl.pallas_call(
        _gate_body,
        out_shape=jax.ShapeDtypeStruct((_B, 1), jnp.float32),
    )(decoder_hidden, context_vector, W1,
      b1.reshape(1, _DEC), jnp.pad(W2, ((0, 127), (0, 0))), b2.reshape(1, 1))

    p_flat = copy_prob.reshape(_B)
    attn_p = jnp.pad(attention_weights, ((0, 0), (0, _SP - _SRC)))
    vocab_p = jnp.pad(vocab_distribution, ((0, 0), (0, _VP - _V)))
    chars_p = jnp.pad(source_chars.astype(jnp.int32),
                      ((0, 0), (0, _SP - _SRC)))

    mesh = plsc.VectorSubcoreMesh(core_axis_name="c", subcore_axis_name="s")
    sc_call = functools.partial(
        pl.kernel, mesh=mesh,
        compiler_params=pltpu.CompilerParams(use_tc_tiling_on_sc=False,
                                             needs_layout_passes=False),
        out_type=jax.ShapeDtypeStruct((_B, _VP), jnp.float32),
        scratch_types=[
            pltpu.VMEM((_L, _VP), jnp.float32),
            pltpu.VMEM((_L, _SP), jnp.float32),
            pltpu.VMEM((_L, _SP), jnp.int32),
            pltpu.VMEM((_L,), jnp.float32),
        ],
    )(_sc_body)
    final_p = sc_call(attn_p, vocab_p, chars_p, p_flat)
    return final_p[:, :_V], copy_prob
